# Initial kernel scaffold; baseline (speedup 1.0000x reference)
import jax
import jax.numpy as jnp
from jax.experimental import pallas as pl


def kernel(x, adj, W1, b1, W2, b2):
    raise NotImplementedError("write your pallas kernel here")


# speedup vs baseline: 11.9730x; 11.9730x over previous
import functools

import jax
import jax.numpy as jnp
from jax import lax
from jax.experimental import pallas as pl
from jax.experimental.pallas import tpu as pltpu
from jax.experimental.pallas import tpu_sc as plsc

N = 10000
NFEAT = 128
NHID = 128
NCLASS = 40
E = 320000

NC = 2
NS = 16
L = 16
NW = NC * NS

IDX_ROWS_PER_TILE = 80
E_PAD = NW * IDX_ROWS_PER_TILE * 128
GROUPS = 5
G = IDX_ROWS_PER_TILE // GROUPS
NP = 10240
RPT = NP // NS
D2P = 48

_mesh = plsc.VectorSubcoreMesh(core_axis_name="c", subcore_axis_name="s")


@functools.partial(
    pl.kernel,
    out_type=jax.ShapeDtypeStruct((NC, NP, L), jnp.float32),
    mesh=_mesh,
    compiler_params=pltpu.CompilerParams(use_tc_tiling_on_sc=False),
    scratch_types=[
        pltpu.VMEM((G, 128), jnp.int32),
        pltpu.VMEM((128, L), jnp.float32),
        pltpu.VMEM_SHARED((NP, L), jnp.float32),
    ],
)
def _sc_degree(col_hbm, out_hbm, cidx, buf, acc):
    c = lax.axis_index("c")
    s = lax.axis_index("s")
    wid = c * NS + s

    @pl.loop(0, 128)
    def _(i):
        buf[i, :] = jnp.zeros((L,), jnp.float32)

    @pl.loop(0, RPT // 128)
    def _(z):
        pltpu.sync_copy(buf, acc.at[pl.ds(s * RPT + z * 128, 128)])

    @pl.loop(0, 128)
    def _(i):
        buf[i, :] = jnp.full((L,), 1.0, jnp.float32)

    plsc.subcore_barrier()

    @pl.loop(0, GROUPS)
    def _(t):
        pltpu.sync_copy(
            col_hbm.at[pl.ds(wid * IDX_ROWS_PER_TILE + t * G, G)], cidx)
        for jj in range(G):
            pltpu.sync_copy(buf, acc.at[cidx.at[jj]], add=True)

    plsc.subcore_barrier()
    pltpu.sync_copy(acc.at[pl.ds(s * RPT, RPT)],
                    out_hbm.at[c, pl.ds(s * RPT, RPT)])


def _make_prop(D):
    @functools.partial(
        pl.kernel,
        out_type=jax.ShapeDtypeStruct((NC, NP, D), jnp.float32),
        mesh=_mesh,
        compiler_params=pltpu.CompilerParams(
            use_tc_tiling_on_sc=(D % 128 == 0)),
        scratch_types=[
            pltpu.VMEM((G, 128), jnp.int32),
            pltpu.VMEM((G, 128), jnp.int32),
            pltpu.VMEM((128, D), jnp.float32),
            pltpu.VMEM_SHARED((NP, D), jnp.float32),
        ],
    )
    def _prop(g_hbm, row_hbm, col_hbm, out_hbm, ridx, cidx, rows, acc):
        c = lax.axis_index("c")
        s = lax.axis_index("s")
        wid = c * NS + s

        pltpu.sync_copy(g_hbm.at[pl.ds(s * RPT, RPT)],
                        acc.at[pl.ds(s * RPT, RPT)])
        plsc.subcore_barrier()

        @pl.loop(0, GROUPS)
        def _(t):
            base = wid * IDX_ROWS_PER_TILE + t * G
            pltpu.sync_copy(row_hbm.at[pl.ds(base, G)], ridx)
            pltpu.sync_copy(col_hbm.at[pl.ds(base, G)], cidx)
            for jj in range(G):
                pltpu.sync_copy(g_hbm.at[ridx.at[jj]], rows)
                pltpu.sync_copy(rows, acc.at[cidx.at[jj]], add=True)

        plsc.subcore_barrier()
        pltpu.sync_copy(acc.at[pl.ds(s * RPT, RPT)],
                        out_hbm.at[c, pl.ds(s * RPT, RPT)])

    return _prop


_prop128 = _make_prop(NHID)
_prop48 = _make_prop(D2P)

BLK = 1024


def _tc_pre_body(d0_ref, d1_ref, x_ref, w_ref, g_ref, dinv_ref):
    deg = d0_ref[...][:, 0:1] + d1_ref[...][:, 0:1] + 1.0
    dinv = lax.rsqrt(deg)
    h = jnp.dot(x_ref[...], w_ref[...], preferred_element_type=jnp.float32)
    g_ref[...] = h * dinv
    dinv_ref[...] = dinv


_tc_pre = pl.pallas_call(
    _tc_pre_body,
    grid=(NP // BLK,),
    in_specs=[
        pl.BlockSpec((BLK, L), lambda i: (i, 0)),
        pl.BlockSpec((BLK, L), lambda i: (i, 0)),
        pl.BlockSpec((BLK, NFEAT), lambda i: (i, 0)),
        pl.BlockSpec((NFEAT, NHID), lambda i: (0, 0)),
    ],
    out_specs=[
        pl.BlockSpec((BLK, NHID), lambda i: (i, 0)),
        pl.BlockSpec((BLK, 1), lambda i: (i, 0)),
    ],
    out_shape=[
        jax.ShapeDtypeStruct((NP, NHID), jnp.float32),
        jax.ShapeDtypeStruct((NP, 1), jnp.float32),
    ],
)


def _tc_mid_body(a_ref, g1_ref, dinv_ref, b1_ref, w2_ref, g2_ref):
    dinv = dinv_ref[...]
    x1 = jnp.maximum(
        dinv * (a_ref[0] + a_ref[1] - g1_ref[...]) + b1_ref[...], 0.0)
    g2_ref[...] = dinv * jnp.dot(
        x1, w2_ref[...], preferred_element_type=jnp.float32)


_tc_mid = pl.pallas_call(
    _tc_mid_body,
    grid=(NP // BLK,),
    in_specs=[
        pl.BlockSpec((NC, BLK, NHID), lambda i: (0, i, 0)),
        pl.BlockSpec((BLK, NHID), lambda i: (i, 0)),
        pl.BlockSpec((BLK, 1), lambda i: (i, 0)),
        pl.BlockSpec((1, NHID), lambda i: (0, 0)),
        pl.BlockSpec((NHID, D2P), lambda i: (0, 0)),
    ],
    out_specs=pl.BlockSpec((BLK, D2P), lambda i: (i, 0)),
    out_shape=jax.ShapeDtypeStruct((NP, D2P), jnp.float32),
)


def _tc_post_body(a_ref, g2_ref, dinv_ref, b2_ref, o_ref):
    dinv = dinv_ref[...]
    o_ref[...] = dinv * (a_ref[0] + a_ref[1] - g2_ref[...]) + b2_ref[...]


_tc_post = pl.pallas_call(
    _tc_post_body,
    grid=(NP // BLK,),
    in_specs=[
        pl.BlockSpec((NC, BLK, D2P), lambda i: (0, i, 0)),
        pl.BlockSpec((BLK, D2P), lambda i: (i, 0)),
        pl.BlockSpec((BLK, 1), lambda i: (i, 0)),
        pl.BlockSpec((1, D2P), lambda i: (0, 0)),
    ],
    out_specs=pl.BlockSpec((BLK, D2P), lambda i: (i, 0)),
    out_shape=jax.ShapeDtypeStruct((NP, D2P), jnp.float32),
)


def kernel(x, adj, W1, b1, W2, b2):
    row = adj[0].astype(jnp.int32)
    col = adj[1].astype(jnp.int32)
    pad = E_PAD - E
    rowp = jnp.concatenate(
        [row, jnp.zeros((pad,), jnp.int32)]).reshape(E_PAD // 128, 128)
    colp = jnp.concatenate(
        [col, jnp.full((pad,), N, jnp.int32)]).reshape(E_PAD // 128, 128)
    xp = jnp.pad(x, ((0, NP - N), (0, 0)))

    degp = _sc_degree(colp)
    g1, dinv = _tc_pre(degp[0], degp[1], xp, W1)
    acc1 = _prop128(g1, rowp, colp)

    W2p = jnp.pad(W2, ((0, 0), (0, D2P - NCLASS)))
    b1r = b1.reshape(1, NHID)
    b2p = jnp.pad(b2, (0, D2P - NCLASS)).reshape(1, D2P)

    g2 = _tc_mid(acc1, g1, dinv, b1r, W2p)
    acc2 = _prop48(g2, rowp, colp)
    out = _tc_post(acc2, g2, dinv, b2p)
    return out[:N, :NCLASS]
